# baseline (speedup 1.0000x reference)
import functools

import jax
import jax.numpy as jnp
from jax import lax
from jax.experimental import pallas as pl
from jax.experimental.pallas import tpu as pltpu
from jax.experimental.pallas import tpu_sc as plsc

N = 10000
F = 128
H1 = 8
C1 = 8
D1 = 64
C2 = 40
NP = 10016
NPA = 10240
RPT = NPA // 16
KC = 128
KR = KC // 128
NT = 32
E_TOT = 320000 + N
NCHUNK = 3 * ((E_TOT + 3 * NT * KC - 1) // (3 * NT * KC))
EPAD = NCHUNK * NT * KC
IDXR = NCHUNK * KR
W1ROW = 32
W2ROW = 32
M1ROW = 80
M2ROW = 48
F32 = jnp.float32
I32 = jnp.int32


def _tc_prep1(x_pad, W1, A_src, A_dst):
    def body(x_ref, w_ref, asr_ref, adr_ref, h_ref, as_ref, ad_ref, m_ref):
        h = jnp.dot(x_ref[...], w_ref[...], preferred_element_type=F32)
        a_s = jnp.dot(h, asr_ref[...], preferred_element_type=F32)
        a_d = jnp.dot(h, adr_ref[...], preferred_element_type=F32)
        h_ref[...] = h
        as_ref[...] = a_s
        ad_ref[...] = a_d
        m = jnp.max(a_s) + jnp.max(a_d)
        m_ref[...] = jnp.full((8, 16), m, F32)

    return pl.pallas_call(
        body,
        out_shape=(
            jax.ShapeDtypeStruct((NP, D1), F32),
            jax.ShapeDtypeStruct((NP, 8), F32),
            jax.ShapeDtypeStruct((NP, 8), F32),
            jax.ShapeDtypeStruct((8, 16), F32),
        ),
    )(x_pad, W1, A_src, A_dst)


def _tc_mid(acc1, b1, W2, a_src2, a_dst2, E8):
    def body(acc_ref, b1_ref, w2_ref, as2_ref, ad2_ref, e8_ref,
             h_ref, as_ref, ad_ref, m_ref):
        num = acc_ref[0, 0:NP, 0:D1] + acc_ref[1, 0:NP, 0:D1]
        den = acc_ref[0, 0:NP, D1:D1 + H1] + acc_ref[1, 0:NP, D1:D1 + H1]
        den_exp = jnp.dot(den, e8_ref[...], preferred_element_type=F32)
        x2 = jax.nn.relu(num / (den_exp + 1e-16) + b1_ref[...])
        rowid = lax.broadcasted_iota(I32, (NP, 1), 0)
        x2 = jnp.where(rowid < N, x2, 0.0)
        h2 = jnp.dot(x2, w2_ref[...], preferred_element_type=F32)
        as2 = jnp.dot(h2, as2_ref[...], preferred_element_type=F32)
        ad2 = jnp.dot(h2, ad2_ref[...], preferred_element_type=F32)
        h_ref[...] = h2
        as_ref[...] = jnp.concatenate([as2, jnp.zeros((NP, 7), F32)], axis=1)
        ad_ref[...] = jnp.concatenate([ad2, jnp.zeros((NP, 7), F32)], axis=1)
        m = jnp.max(as2) + jnp.max(ad2)
        m_ref[...] = jnp.full((8, 16), m, F32)

    return pl.pallas_call(
        body,
        out_shape=(
            jax.ShapeDtypeStruct((NP, C2), F32),
            jax.ShapeDtypeStruct((NP, 8), F32),
            jax.ShapeDtypeStruct((NP, 8), F32),
            jax.ShapeDtypeStruct((8, 16), F32),
        ),
    )(acc1, b1, W2, a_src2, a_dst2, E8)


def _tc_final(acc2, b2):
    def body(acc_ref, b2_ref, out_ref):
        num = acc_ref[0, 0:N, 0:C2] + acc_ref[1, 0:N, 0:C2]
        den = acc_ref[0, 0:N, C2:C2 + 1] + acc_ref[1, 0:N, C2:C2 + 1]
        o = num / (den + 1e-16) + b2_ref[...]
        mx = jnp.max(o, axis=1, keepdims=True)
        z = o - mx
        out_ref[...] = z - jnp.log(jnp.sum(jnp.exp(z), axis=1, keepdims=True))

    return pl.pallas_call(
        body,
        out_shape=jax.ShapeDtypeStruct((N, C2), F32),
    )(acc2, b2)


def _make_sc_edge_pass(C, H, WROW, MROW, split_as):
    mesh = plsc.VectorSubcoreMesh(core_axis_name="c", subcore_axis_name="s")
    CH = C * H
    NPAIR = CH // 2

    @functools.partial(
        pl.kernel,
        out_type=jax.ShapeDtypeStruct((2, NPA, MROW), F32),
        mesh=mesh,
        compiler_params=pltpu.CompilerParams(
            needs_layout_passes=False, use_tc_tiling_on_sc=False),
        scratch_types=[
            pltpu.VMEM((IDXR, 128), I32),
            pltpu.VMEM((IDXR, 128), I32),
            pltpu.VMEM((KC, WROW), F32),
            pltpu.VMEM((KC, WROW), F32),
            pltpu.VMEM((KC, WROW), F32),
            pltpu.VMEM((KC, MROW), F32),
            pltpu.VMEM((KC, MROW), F32),
            pltpu.VMEM((KC, MROW), F32),
            pltpu.VMEM((KC, 8), F32),
            pltpu.VMEM((KC, 8), F32),
            pltpu.VMEM((KC, 8), F32),
            pltpu.VMEM((KC, 8), F32),
            pltpu.VMEM((KC, 8), F32),
            pltpu.VMEM((KC, 8), F32),
            pltpu.VMEM((16,), F32),
            pltpu.VMEM_SHARED((NPA, MROW), F32),
            pltpu.SemaphoreType.DMA,
            pltpu.SemaphoreType.DMA,
            pltpu.SemaphoreType.DMA,
            pltpu.SemaphoreType.DMA,
            pltpu.SemaphoreType.DMA,
            pltpu.SemaphoreType.DMA,
        ],
    )
    def sc_pass(tsrc, astab, adtab, m, src, dst, zrows, out,
                sloc, dloc, rows0, rows1, rows2, msg0, msg1, msg2,
                ad0, ad1, ad2, as0, as1, as2, mbuf, acc,
                semg0, semg1, semg2, sems0, sems1, sems2):
        cid = lax.axis_index("c")
        sid = lax.axis_index("s")
        wid = cid * 16 + sid
        rows = (rows0, rows1, rows2)
        msg = (msg0, msg1, msg2)
        ad = (ad0, ad1, ad2)
        asb = (as0, as1, as2)
        semg = (semg0, semg1, semg2)
        sems = (sems0, sems1, sems2)
        pltpu.sync_copy(zrows, acc.at[pl.ds(sid * RPT, RPT)])
        pltpu.sync_copy(m.at[0], mbuf)
        pltpu.sync_copy(src.at[pl.ds(wid * IDXR, IDXR)], sloc)
        pltpu.sync_copy(dst.at[pl.ds(wid * IDXR, IDXR)], dloc)
        plsc.subcore_barrier()

        iota = lax.iota(I32, 16)
        m_v = mbuf[...]

        def issue_gathers(ci, b):
            for j in range(KR):
                pltpu.async_copy(tsrc.at[sloc.at[ci * KR + j]],
                                 rows[b].at[pl.ds(j * 128, 128)], semg[b])
                pltpu.async_copy(adtab.at[dloc.at[ci * KR + j]],
                                 ad[b].at[pl.ds(j * 128, 128)], semg[b])
                if split_as:
                    pltpu.async_copy(astab.at[sloc.at[ci * KR + j]],
                                     asb[b].at[pl.ds(j * 128, 128)], semg[b])

        def wait_gathers(b):
            pltpu.make_async_copy(tsrc.at[pl.ds(0, KC)], rows[b], semg[b]).wait()
            pltpu.make_async_copy(adtab.at[pl.ds(0, KC)], ad[b], semg[b]).wait()
            if split_as:
                pltpu.make_async_copy(astab.at[pl.ds(0, KC)], asb[b],
                                      semg[b]).wait()

        def issue_scatter(ci, b):
            for j in range(KR):
                pltpu.async_copy(msg[b].at[pl.ds(j * 128, 128)],
                                 acc.at[dloc.at[ci * KR + j]], sems[b], add=True)

        def wait_scatter(b):
            pltpu.make_async_copy(msg[b], acc.at[pl.ds(0, KC)], sems[b]).wait()

        def compute(b):
            rbuf = rows[b]
            mgb = msg[b]
            abuf = ad[b]
            sbuf = asb[b]

            def group(g, _):
                rowid = g * 16 + iota
                ws = []
                for h in range(H):
                    if split_as:
                        as_v = plsc.load_gather(
                            sbuf, [rowid, jnp.full((16,), h, I32)])
                    else:
                        as_v = plsc.load_gather(
                            rbuf, [rowid, jnp.full((16,), NPAIR + h, I32)])
                    ad_v = plsc.load_gather(
                        abuf, [rowid, jnp.full((16,), h, I32)])
                    e = as_v + ad_v
                    e = jnp.where(e >= 0.0, e, e * 0.2)
                    w = jnp.exp(e - m_v)
                    plsc.store_scatter(
                        mgb, [rowid, jnp.full((16,), CH + h, I32)], w)
                    ws.append(w)
                for cp in range(NPAIR):
                    u = plsc.load_gather(rbuf, [rowid, jnp.full((16,), cp, I32)])
                    ub = plsc.bitcast(u, I32)
                    lo = plsc.bitcast(jnp.left_shift(ub, 16), F32)
                    hi = plsc.bitcast(jnp.bitwise_and(ub, jnp.int32(-65536)), F32)
                    wv = ws[(2 * cp) // C]
                    plsc.store_scatter(
                        mgb, [rowid, jnp.full((16,), 2 * cp, I32)], lo * wv)
                    plsc.store_scatter(
                        mgb, [rowid, jnp.full((16,), 2 * cp + 1, I32)], hi * wv)
                return 0

            lax.fori_loop(0, KC // 16, group, 0)

        def phase(ci, b):
            nb = (b + 1) % 3

            @pl.when(ci >= 2)
            def _():
                wait_scatter(nb)

            @pl.when(ci + 2 < NCHUNK)
            def _():
                issue_gathers(ci + 2, (b + 2) % 3)

            wait_gathers(b)
            compute(b)
            issue_scatter(ci, b)

        issue_gathers(0, 0)
        issue_gathers(1, 1)

        def triple_body(t, _):
            phase(3 * t, 0)
            phase(3 * t + 1, 1)
            phase(3 * t + 2, 2)
            return 0

        lax.fori_loop(0, NCHUNK // 3, triple_body, 0)
        wait_scatter((NCHUNK - 2) % 3)
        wait_scatter((NCHUNK - 1) % 3)
        plsc.subcore_barrier()
        pltpu.sync_copy(acc.at[pl.ds(sid * RPT, RPT)],
                        out.at[cid, pl.ds(sid * RPT, RPT)])

    return sc_pass


_sc_pass1 = _make_sc_edge_pass(C1, H1, W1ROW, M1ROW, True)
_sc_pass2 = _make_sc_edge_pass(C2, 1, W2ROW, M2ROW, False)


def _pack_bf16(h):
    hb = h.astype(jnp.bfloat16).reshape(h.shape[0], h.shape[1] // 2, 2)
    return jax.lax.bitcast_convert_type(hb, F32)


def kernel(x, edge_index, W1, a_src1, a_dst1, b1, W2, a_src2, a_dst2, b2):
    ei = edge_index.astype(I32)
    loop = jnp.arange(N, dtype=I32)
    padn = EPAD - E_TOT
    src = jnp.concatenate([ei[0], loop, jnp.full((padn,), N, I32)])
    dst = jnp.concatenate([ei[1], loop, jnp.full((padn,), N, I32)])
    src = src.reshape(EPAD // 128, 128)
    dst = dst.reshape(EPAD // 128, 128)
    x_pad = jnp.pad(x, ((0, NP - N), (0, 0)))
    eye8 = jnp.eye(8, dtype=F32)
    A_src1 = (a_src1.reshape(H1, C1)[:, :, None] * eye8[:, None, :]).reshape(D1, H1)
    A_dst1 = (a_dst1.reshape(H1, C1)[:, :, None] * eye8[:, None, :]).reshape(D1, H1)
    E8 = jnp.kron(eye8, jnp.ones((1, C1), F32))
    z1 = jnp.zeros((RPT, M1ROW), F32)
    z2 = jnp.zeros((RPT, M2ROW), F32)

    h1, as1, ad1t, m1 = _tc_prep1(x_pad, W1, A_src1, A_dst1)
    tsrc1 = _pack_bf16(h1)
    acc1 = _sc_pass1(tsrc1, as1, ad1t, m1, src, dst, z1)
    h2, as2t, ad2t, m2 = _tc_mid(acc1, b1.reshape(1, D1), W2,
                                 a_src2.reshape(C2, 1), a_dst2.reshape(C2, 1), E8)
    tsrc2 = jnp.concatenate(
        [_pack_bf16(h2), as2t[:, 0:1], jnp.zeros((NP, 11), F32)], 1)
    acc2 = _sc_pass2(tsrc2, as2t, ad2t, m2, src, dst, z2)
    return _tc_final(acc2, b2.reshape(1, C2))


# speedup vs baseline: 1.1061x; 1.1061x over previous
import functools

import jax
import jax.numpy as jnp
from jax import lax
from jax.experimental import pallas as pl
from jax.experimental.pallas import tpu as pltpu
from jax.experimental.pallas import tpu_sc as plsc

N = 10000
F = 128
H1 = 8
C1 = 8
D1 = 64
C2 = 40
NP = 10016
NPA = 10240
RPT = NPA // 16
KC = 128
KR = KC // 128
NT = 32
E_TOT = 320000 + N
NCHUNK = 3 * ((E_TOT + 3 * NT * KC - 1) // (3 * NT * KC))
EPAD = NCHUNK * NT * KC
IDXR = NCHUNK * KR
W1ROW = 48
W2ROW = 32
M1ROW = 80
M2ROW = 48
F32 = jnp.float32
I32 = jnp.int32


def _tc_prep1(x_pad, W1, A_src, A_dst):
    def body(x_ref, w_ref, asr_ref, adr_ref, h_ref, as_ref, ad_ref, m_ref):
        h = jnp.dot(x_ref[...], w_ref[...], preferred_element_type=F32)
        a_s = jnp.dot(h, asr_ref[...], preferred_element_type=F32)
        a_d = jnp.dot(h, adr_ref[...], preferred_element_type=F32)
        h_ref[...] = h
        as_ref[...] = a_s
        ad_ref[...] = a_d
        m = jnp.max(a_s) + jnp.max(a_d)
        m_ref[...] = jnp.full((8, 16), m, F32)

    return pl.pallas_call(
        body,
        out_shape=(
            jax.ShapeDtypeStruct((NP, D1), F32),
            jax.ShapeDtypeStruct((NP, 8), F32),
            jax.ShapeDtypeStruct((NP, 8), F32),
            jax.ShapeDtypeStruct((8, 16), F32),
        ),
    )(x_pad, W1, A_src, A_dst)


def _tc_mid(acc1, b1, W2, a_src2, a_dst2, E8):
    def body(acc_ref, b1_ref, w2_ref, as2_ref, ad2_ref, e8_ref,
             h_ref, as_ref, ad_ref, m_ref):
        num = acc_ref[0, 0:NP, 0:D1] + acc_ref[1, 0:NP, 0:D1]
        den = acc_ref[0, 0:NP, D1:D1 + H1] + acc_ref[1, 0:NP, D1:D1 + H1]
        den_exp = jnp.dot(den, e8_ref[...], preferred_element_type=F32)
        x2 = jax.nn.relu(num / (den_exp + 1e-16) + b1_ref[...])
        rowid = lax.broadcasted_iota(I32, (NP, 1), 0)
        x2 = jnp.where(rowid < N, x2, 0.0)
        h2 = jnp.dot(x2, w2_ref[...], preferred_element_type=F32)
        as2 = jnp.dot(h2, as2_ref[...], preferred_element_type=F32)
        ad2 = jnp.dot(h2, ad2_ref[...], preferred_element_type=F32)
        h_ref[...] = h2
        as_ref[...] = jnp.concatenate([as2, jnp.zeros((NP, 7), F32)], axis=1)
        ad_ref[...] = jnp.concatenate([ad2, jnp.zeros((NP, 7), F32)], axis=1)
        m = jnp.max(as2) + jnp.max(ad2)
        m_ref[...] = jnp.full((8, 16), m, F32)

    return pl.pallas_call(
        body,
        out_shape=(
            jax.ShapeDtypeStruct((NP, C2), F32),
            jax.ShapeDtypeStruct((NP, 8), F32),
            jax.ShapeDtypeStruct((NP, 8), F32),
            jax.ShapeDtypeStruct((8, 16), F32),
        ),
    )(acc1, b1, W2, a_src2, a_dst2, E8)


def _tc_final(acc2, b2):
    def body(acc_ref, b2_ref, out_ref):
        num = acc_ref[0, 0:N, 0:C2] + acc_ref[1, 0:N, 0:C2]
        den = acc_ref[0, 0:N, C2:C2 + 1] + acc_ref[1, 0:N, C2:C2 + 1]
        o = num / (den + 1e-16) + b2_ref[...]
        mx = jnp.max(o, axis=1, keepdims=True)
        z = o - mx
        out_ref[...] = z - jnp.log(jnp.sum(jnp.exp(z), axis=1, keepdims=True))

    return pl.pallas_call(
        body,
        out_shape=jax.ShapeDtypeStruct((N, C2), F32),
    )(acc2, b2)


def _make_sc_edge_pass(C, H, WROW, MROW, split_as):
    mesh = plsc.VectorSubcoreMesh(core_axis_name="c", subcore_axis_name="s")
    CH = C * H
    NPAIR = CH // 2

    @functools.partial(
        pl.kernel,
        out_type=jax.ShapeDtypeStruct((2, NPA, MROW), F32),
        mesh=mesh,
        compiler_params=pltpu.CompilerParams(
            needs_layout_passes=False, use_tc_tiling_on_sc=False),
        scratch_types=[
            pltpu.VMEM((IDXR, 128), I32),
            pltpu.VMEM((IDXR, 128), I32),
            pltpu.VMEM((KC, WROW), F32),
            pltpu.VMEM((KC, WROW), F32),
            pltpu.VMEM((KC, WROW), F32),
            pltpu.VMEM((KC, MROW), F32),
            pltpu.VMEM((KC, MROW), F32),
            pltpu.VMEM((KC, MROW), F32),
            pltpu.VMEM((KC, 8), F32),
            pltpu.VMEM((KC, 8), F32),
            pltpu.VMEM((KC, 8), F32),
            pltpu.VMEM((KC, 8), F32),
            pltpu.VMEM((KC, 8), F32),
            pltpu.VMEM((KC, 8), F32),
            pltpu.VMEM((16,), F32),
            pltpu.VMEM_SHARED((NPA, MROW), F32),
            pltpu.SemaphoreType.DMA,
            pltpu.SemaphoreType.DMA,
            pltpu.SemaphoreType.DMA,
            pltpu.SemaphoreType.DMA,
            pltpu.SemaphoreType.DMA,
            pltpu.SemaphoreType.DMA,
        ],
    )
    def sc_pass(tsrc, astab, adtab, m, src, dst, zrows, out,
                sloc, dloc, rows0, rows1, rows2, msg0, msg1, msg2,
                ad0, ad1, ad2, as0, as1, as2, mbuf, acc,
                semg0, semg1, semg2, sems0, sems1, sems2):
        cid = lax.axis_index("c")
        sid = lax.axis_index("s")
        wid = cid * 16 + sid
        rows = (rows0, rows1, rows2)
        msg = (msg0, msg1, msg2)
        ad = (ad0, ad1, ad2)
        asb = (as0, as1, as2)
        semg = (semg0, semg1, semg2)
        sems = (sems0, sems1, sems2)
        pltpu.sync_copy(zrows, acc.at[pl.ds(sid * RPT, RPT)])
        pltpu.sync_copy(m.at[0], mbuf)
        pltpu.sync_copy(src.at[pl.ds(wid * IDXR, IDXR)], sloc)
        pltpu.sync_copy(dst.at[pl.ds(wid * IDXR, IDXR)], dloc)
        plsc.subcore_barrier()

        iota = lax.iota(I32, 16)
        m_v = mbuf[...]

        def issue_gathers(ci, b):
            for j in range(KR):
                pltpu.async_copy(tsrc.at[sloc.at[ci * KR + j]],
                                 rows[b].at[pl.ds(j * 128, 128)], semg[b])
                pltpu.async_copy(adtab.at[dloc.at[ci * KR + j]],
                                 ad[b].at[pl.ds(j * 128, 128)], semg[b])
                if split_as:
                    pltpu.async_copy(astab.at[sloc.at[ci * KR + j]],
                                     asb[b].at[pl.ds(j * 128, 128)], semg[b])

        def wait_gathers(b):
            pltpu.make_async_copy(tsrc.at[pl.ds(0, KC)], rows[b], semg[b]).wait()
            pltpu.make_async_copy(adtab.at[pl.ds(0, KC)], ad[b], semg[b]).wait()
            if split_as:
                pltpu.make_async_copy(astab.at[pl.ds(0, KC)], asb[b],
                                      semg[b]).wait()

        def issue_scatter(ci, b):
            for j in range(KR):
                pltpu.async_copy(msg[b].at[pl.ds(j * 128, 128)],
                                 acc.at[dloc.at[ci * KR + j]], sems[b], add=True)

        def wait_scatter(b):
            pltpu.make_async_copy(msg[b], acc.at[pl.ds(0, KC)], sems[b]).wait()

        def compute(b):
            rbuf = rows[b]
            mgb = msg[b]
            abuf = ad[b]
            sbuf = asb[b]

            def group(g, _):
                rowid = g * 16 + iota
                ws = []
                for h in range(H):
                    if split_as:
                        as_v = plsc.load_gather(
                            sbuf, [rowid, jnp.full((16,), h, I32)])
                    else:
                        as_v = plsc.load_gather(
                            rbuf, [rowid, jnp.full((16,), NPAIR + h, I32)])
                    ad_v = plsc.load_gather(
                        abuf, [rowid, jnp.full((16,), h, I32)])
                    e = as_v + ad_v
                    e = jnp.where(e >= 0.0, e, e * 0.2)
                    w = jnp.exp(e - m_v)
                    plsc.store_scatter(
                        mgb, [rowid, jnp.full((16,), CH + h, I32)], w)
                    ws.append(w)
                for cp in range(NPAIR):
                    u = plsc.load_gather(rbuf, [rowid, jnp.full((16,), cp, I32)])
                    ub = plsc.bitcast(u, I32)
                    lo = plsc.bitcast(jnp.left_shift(ub, 16), F32)
                    hi = plsc.bitcast(jnp.bitwise_and(ub, jnp.int32(-65536)), F32)
                    wv = ws[(2 * cp) // C]
                    plsc.store_scatter(
                        mgb, [rowid, jnp.full((16,), 2 * cp, I32)], lo * wv)
                    plsc.store_scatter(
                        mgb, [rowid, jnp.full((16,), 2 * cp + 1, I32)], hi * wv)
                return 0

            lax.fori_loop(0, KC // 16, group, 0)

        def phase(ci, b):
            nb = (b + 1) % 3

            @pl.when(ci >= 2)
            def _():
                wait_scatter(nb)

            @pl.when(ci + 2 < NCHUNK)
            def _():
                issue_gathers(ci + 2, (b + 2) % 3)

            wait_gathers(b)
            compute(b)
            issue_scatter(ci, b)

        issue_gathers(0, 0)
        issue_gathers(1, 1)

        def triple_body(t, _):
            phase(3 * t, 0)
            phase(3 * t + 1, 1)
            phase(3 * t + 2, 2)
            return 0

        lax.fori_loop(0, NCHUNK // 3, triple_body, 0)
        wait_scatter((NCHUNK - 2) % 3)
        wait_scatter((NCHUNK - 1) % 3)
        plsc.subcore_barrier()
        pltpu.sync_copy(acc.at[pl.ds(sid * RPT, RPT)],
                        out.at[cid, pl.ds(sid * RPT, RPT)])

    return sc_pass


_sc_pass1 = _make_sc_edge_pass(C1, H1, W1ROW, M1ROW, False)
_sc_pass2 = _make_sc_edge_pass(C2, 1, W2ROW, M2ROW, False)


def _pack_bf16(h):
    hb = h.astype(jnp.bfloat16).reshape(h.shape[0], h.shape[1] // 2, 2)
    return jax.lax.bitcast_convert_type(hb, F32)


def kernel(x, edge_index, W1, a_src1, a_dst1, b1, W2, a_src2, a_dst2, b2):
    ei = edge_index.astype(I32)
    loop = jnp.arange(N, dtype=I32)
    padn = EPAD - E_TOT
    src = jnp.concatenate([ei[0], loop, jnp.full((padn,), N, I32)])
    dst = jnp.concatenate([ei[1], loop, jnp.full((padn,), N, I32)])
    src = src.reshape(EPAD // 128, 128)
    dst = dst.reshape(EPAD // 128, 128)
    x_pad = jnp.pad(x, ((0, NP - N), (0, 0)))
    eye8 = jnp.eye(8, dtype=F32)
    A_src1 = (a_src1.reshape(H1, C1)[:, :, None] * eye8[:, None, :]).reshape(D1, H1)
    A_dst1 = (a_dst1.reshape(H1, C1)[:, :, None] * eye8[:, None, :]).reshape(D1, H1)
    E8 = jnp.kron(eye8, jnp.ones((1, C1), F32))
    z1 = jnp.zeros((RPT, M1ROW), F32)
    z2 = jnp.zeros((RPT, M2ROW), F32)

    h1, as1, ad1t, m1 = _tc_prep1(x_pad, W1, A_src1, A_dst1)
    tsrc1 = jnp.concatenate([_pack_bf16(h1), as1, jnp.zeros((NP, 8), F32)], 1)
    acc1 = _sc_pass1(tsrc1, as1, ad1t, m1, src, dst, z1)
    h2, as2t, ad2t, m2 = _tc_mid(acc1, b1.reshape(1, D1), W2,
                                 a_src2.reshape(C2, 1), a_dst2.reshape(C2, 1), E8)
    tsrc2 = jnp.concatenate(
        [_pack_bf16(h2), as2t[:, 0:1], jnp.zeros((NP, 11), F32)], 1)
    acc2 = _sc_pass2(tsrc2, as2t, ad2t, m2, src, dst, z2)
    return _tc_final(acc2, b2.reshape(1, C2))


# speedup vs baseline: 1.1231x; 1.0153x over previous
import functools

import jax
import jax.numpy as jnp
from jax import lax
from jax.experimental import pallas as pl
from jax.experimental.pallas import tpu as pltpu
from jax.experimental.pallas import tpu_sc as plsc

N = 10000
F = 128
H1 = 8
C1 = 8
D1 = 64
C2 = 40
NP = 10016
NPA = 10240
RPT = NPA // 16
KC = 128
KR = KC // 128
NT = 32
E_TOT = 320000 + N
NCHUNK = 3 * ((E_TOT + 3 * NT * KC - 1) // (3 * NT * KC))
EPAD = NCHUNK * NT * KC
IDXR = NCHUNK * KR
W1ROW = 48
W2ROW = 32
M1ROW = 80
M2ROW = 48
F32 = jnp.float32
I32 = jnp.int32


def _tc_prep1(x_pad, W1, A_src, A_dst):
    def body(x_ref, w_ref, asr_ref, adr_ref, h_ref, as_ref, ad_ref, m_ref):
        h = jnp.dot(x_ref[...], w_ref[...], preferred_element_type=F32)
        a_s = jnp.dot(h, asr_ref[...], preferred_element_type=F32)
        a_d = jnp.dot(h, adr_ref[...], preferred_element_type=F32)
        h_ref[...] = h
        as_ref[...] = a_s
        ad_ref[...] = a_d
        m = jnp.max(a_s) + jnp.max(a_d)
        m_ref[...] = jnp.full((8, 16), m, F32)

    return pl.pallas_call(
        body,
        out_shape=(
            jax.ShapeDtypeStruct((NP, D1), F32),
            jax.ShapeDtypeStruct((NP, 8), F32),
            jax.ShapeDtypeStruct((NP, 8), F32),
            jax.ShapeDtypeStruct((8, 16), F32),
        ),
    )(x_pad, W1, A_src, A_dst)


def _tc_mid(acc1, b1, W2, a_src2, a_dst2, E8):
    def body(acc_ref, b1_ref, w2_ref, as2_ref, ad2_ref, e8_ref,
             h_ref, as_ref, ad_ref, m_ref):
        num = acc_ref[0, 0:NP, 0:D1] + acc_ref[1, 0:NP, 0:D1]
        den = acc_ref[0, 0:NP, D1:D1 + H1] + acc_ref[1, 0:NP, D1:D1 + H1]
        den_exp = jnp.dot(den, e8_ref[...], preferred_element_type=F32)
        x2 = jax.nn.relu(num / (den_exp + 1e-16) + b1_ref[...])
        rowid = lax.broadcasted_iota(I32, (NP, 1), 0)
        x2 = jnp.where(rowid < N, x2, 0.0)
        h2 = jnp.dot(x2, w2_ref[...], preferred_element_type=F32)
        as2 = jnp.dot(h2, as2_ref[...], preferred_element_type=F32)
        ad2 = jnp.dot(h2, ad2_ref[...], preferred_element_type=F32)
        h_ref[...] = h2
        as_ref[...] = jnp.concatenate([as2, jnp.zeros((NP, 7), F32)], axis=1)
        ad_ref[...] = jnp.concatenate([ad2, jnp.zeros((NP, 7), F32)], axis=1)
        m = jnp.max(as2) + jnp.max(ad2)
        m_ref[...] = jnp.full((8, 16), m, F32)

    return pl.pallas_call(
        body,
        out_shape=(
            jax.ShapeDtypeStruct((NP, C2), F32),
            jax.ShapeDtypeStruct((NP, 8), F32),
            jax.ShapeDtypeStruct((NP, 8), F32),
            jax.ShapeDtypeStruct((8, 16), F32),
        ),
    )(acc1, b1, W2, a_src2, a_dst2, E8)


def _tc_final(acc2, b2):
    def body(acc_ref, b2_ref, out_ref):
        num = acc_ref[0, 0:N, 0:C2] + acc_ref[1, 0:N, 0:C2]
        den = acc_ref[0, 0:N, C2:C2 + 1] + acc_ref[1, 0:N, C2:C2 + 1]
        o = num / (den + 1e-16) + b2_ref[...]
        mx = jnp.max(o, axis=1, keepdims=True)
        z = o - mx
        out_ref[...] = z - jnp.log(jnp.sum(jnp.exp(z), axis=1, keepdims=True))

    return pl.pallas_call(
        body,
        out_shape=jax.ShapeDtypeStruct((N, C2), F32),
    )(acc2, b2)


def _make_sc_edge_pass(C, H, WROW, MROW):
    mesh = plsc.VectorSubcoreMesh(core_axis_name="c", subcore_axis_name="s")
    CH = C * H
    NPAIR = CH // 2

    @functools.partial(
        pl.kernel,
        out_type=jax.ShapeDtypeStruct((2, NPA, MROW), F32),
        mesh=mesh,
        compiler_params=pltpu.CompilerParams(
            needs_layout_passes=False, use_tc_tiling_on_sc=False),
        scratch_types=[
            pltpu.VMEM((IDXR, 128), I32),
            pltpu.VMEM((IDXR, 128), I32),
            pltpu.VMEM((KC, WROW), F32),
            pltpu.VMEM((KC, WROW), F32),
            pltpu.VMEM((KC, WROW), F32),
            pltpu.VMEM((KC, MROW), F32),
            pltpu.VMEM((KC, MROW), F32),
            pltpu.VMEM((KC, MROW), F32),
            pltpu.VMEM((KC, 8), F32),
            pltpu.VMEM((KC, 8), F32),
            pltpu.VMEM((KC, 8), F32),
            pltpu.VMEM((16,), F32),
            pltpu.VMEM_SHARED((NPA, MROW), F32),
            pltpu.SemaphoreType.DMA,
            pltpu.SemaphoreType.DMA,
            pltpu.SemaphoreType.DMA,
            pltpu.SemaphoreType.DMA,
            pltpu.SemaphoreType.DMA,
            pltpu.SemaphoreType.DMA,
        ],
    )
    def sc_pass(tsrc, adtab, m, src, dst, zrows, out,
                sloc, dloc, rows0, rows1, rows2, msg0, msg1, msg2,
                ad0, ad1, ad2, mbuf, acc,
                semg0, semg1, semg2, sems0, sems1, sems2):
        cid = lax.axis_index("c")
        sid = lax.axis_index("s")
        wid = cid * 16 + sid
        rows = (rows0, rows1, rows2)
        msg = (msg0, msg1, msg2)
        ad = (ad0, ad1, ad2)
        semg = (semg0, semg1, semg2)
        sems = (sems0, sems1, sems2)
        pltpu.sync_copy(zrows, acc.at[pl.ds(sid * RPT, RPT)])
        pltpu.sync_copy(m.at[0], mbuf)
        pltpu.sync_copy(src.at[pl.ds(wid * IDXR, IDXR)], sloc)
        pltpu.sync_copy(dst.at[pl.ds(wid * IDXR, IDXR)], dloc)
        plsc.subcore_barrier()

        iota = lax.iota(I32, 16)
        m_v = mbuf[...]

        def issue_gathers(ci, b):
            for j in range(KR):
                pltpu.async_copy(tsrc.at[sloc.at[ci * KR + j]],
                                 rows[b].at[pl.ds(j * 128, 128)], semg[b])
                pltpu.async_copy(adtab.at[dloc.at[ci * KR + j]],
                                 ad[b].at[pl.ds(j * 128, 128)], semg[b])

        def wait_gathers(b):
            pltpu.make_async_copy(tsrc.at[pl.ds(0, KC)], rows[b], semg[b]).wait()
            pltpu.make_async_copy(adtab.at[pl.ds(0, KC)], ad[b], semg[b]).wait()

        def issue_scatter(ci, b):
            for j in range(KR):
                pltpu.async_copy(msg[b].at[pl.ds(j * 128, 128)],
                                 acc.at[dloc.at[ci * KR + j]], sems[b], add=True)

        def wait_scatter(b):
            pltpu.make_async_copy(msg[b], acc.at[pl.ds(0, KC)], sems[b]).wait()

        def compute(b):
            rbuf = rows[b]
            mgb = msg[b]
            abuf = ad[b]

            def group(g, _):
                rowid = g * 16 + iota
                ws = []
                for h in range(H):
                    as_v = plsc.load_gather(
                        rbuf, [rowid, jnp.full((16,), NPAIR + h, I32)])
                    ad_v = plsc.load_gather(
                        abuf, [rowid, jnp.full((16,), h, I32)])
                    e = as_v + ad_v
                    e = jnp.where(e >= 0.0, e, e * 0.2)
                    w = jnp.exp(e - m_v)
                    plsc.store_scatter(
                        mgb, [rowid, jnp.full((16,), CH + h, I32)], w)
                    ws.append(w)
                for cp in range(NPAIR):
                    u = plsc.load_gather(rbuf, [rowid, jnp.full((16,), cp, I32)])
                    ub = plsc.bitcast(u, I32)
                    lo = plsc.bitcast(jnp.left_shift(ub, 16), F32)
                    hi = plsc.bitcast(jnp.bitwise_and(ub, jnp.int32(-65536)), F32)
                    wv = ws[(2 * cp) // C]
                    plsc.store_scatter(
                        mgb, [rowid, jnp.full((16,), 2 * cp, I32)], lo * wv)
                    plsc.store_scatter(
                        mgb, [rowid, jnp.full((16,), 2 * cp + 1, I32)], hi * wv)
                return 0

            lax.fori_loop(0, KC // 16, group, 0)

        def phase(ci, b):
            nb = (b + 1) % 3

            @pl.when(ci >= 2)
            def _():
                wait_scatter(nb)

            @pl.when(ci + 2 < NCHUNK)
            def _():
                issue_gathers(ci + 2, (b + 2) % 3)

            wait_gathers(b)
            compute(b)
            issue_scatter(ci, b)

        issue_gathers(0, 0)
        issue_gathers(1, 1)

        def triple_body(t, _):
            phase(3 * t, 0)
            phase(3 * t + 1, 1)
            phase(3 * t + 2, 2)
            return 0

        lax.fori_loop(0, NCHUNK // 3, triple_body, 0)
        wait_scatter((NCHUNK - 2) % 3)
        wait_scatter((NCHUNK - 1) % 3)
        plsc.subcore_barrier()
        pltpu.sync_copy(acc.at[pl.ds(sid * RPT, RPT)],
                        out.at[cid, pl.ds(sid * RPT, RPT)])

    return sc_pass


_sc_pass1 = _make_sc_edge_pass(C1, H1, W1ROW, M1ROW)
_sc_pass2 = _make_sc_edge_pass(C2, 1, W2ROW, M2ROW)


def _pack_bf16(h):
    hb = h.astype(jnp.bfloat16).reshape(h.shape[0], h.shape[1] // 2, 2)
    return jax.lax.bitcast_convert_type(hb, F32)


def kernel(x, edge_index, W1, a_src1, a_dst1, b1, W2, a_src2, a_dst2, b2):
    ei = edge_index.astype(I32)
    loop = jnp.arange(N, dtype=I32)
    padn = EPAD - E_TOT
    src = jnp.concatenate([ei[0], loop, jnp.full((padn,), N, I32)])
    dst = jnp.concatenate([ei[1], loop, jnp.full((padn,), N, I32)])
    src = src.reshape(EPAD // 128, 128)
    dst = dst.reshape(EPAD // 128, 128)
    x_pad = jnp.pad(x, ((0, NP - N), (0, 0)))
    eye8 = jnp.eye(8, dtype=F32)
    A_src1 = (a_src1.reshape(H1, C1)[:, :, None] * eye8[:, None, :]).reshape(D1, H1)
    A_dst1 = (a_dst1.reshape(H1, C1)[:, :, None] * eye8[:, None, :]).reshape(D1, H1)
    E8 = jnp.kron(eye8, jnp.ones((1, C1), F32))
    z1 = jnp.zeros((RPT, M1ROW), F32)
    z2 = jnp.zeros((RPT, M2ROW), F32)

    h1, as1, ad1t, m1 = _tc_prep1(x_pad, W1, A_src1, A_dst1)
    tsrc1 = jnp.concatenate([_pack_bf16(h1), as1, jnp.zeros((NP, 8), F32)], 1)
    acc1 = _sc_pass1(tsrc1, ad1t, m1, src, dst, z1)
    h2, as2t, ad2t, m2 = _tc_mid(acc1, b1.reshape(1, D1), W2,
                                 a_src2.reshape(C2, 1), a_dst2.reshape(C2, 1), E8)
    tsrc2 = jnp.concatenate(
        [_pack_bf16(h2), as2t[:, 0:1], jnp.zeros((NP, 11), F32)], 1)
    acc2 = _sc_pass2(tsrc2, ad2t, m2, src, dst, z2)
    return _tc_final(acc2, b2.reshape(1, C2))
